# baseline (speedup 1.0000x reference)
import jax
import jax.numpy as jnp
from jax import lax
from jax.experimental import pallas as pl
from jax.experimental.pallas import tpu as pltpu
from jax.experimental.pallas import tpu_sc as plsc

NUM_EMB = 1000000
D = 32
BATCH = 16384
HIST = 20
NC = 2
NS = 16
NW = NC * NS
HB = BATCH // 128
BT_PER_W = HB // NW
NBLK = HIST * BT_PER_W


def _transpose_block(buf, tbuf, iota16):
    for c in range(D):
        col = jnp.full((16,), c, jnp.int32)
        for k in range(8):
            vals = plsc.load_gather(buf, [iota16 + (16 * k), col])
            tbuf[c // 8, c % 8, pl.ds(16 * k, 16)] = vals


def _emb_body(xt_hbm, table_hbm, out_hbm, idx_v, buf_a, buf_b, tbuf_a, tbuf_b,
              sem_a, sem_b, sem_sa, sem_sb):
    w = lax.axis_index("s") * NC + lax.axis_index("c")
    bt0 = w * BT_PER_W
    pltpu.sync_copy(xt_hbm.at[:, pl.ds(bt0, BT_PER_W)], idx_v)
    iota16 = lax.iota(jnp.int32, 16)

    def fire(blk, buf, sem):
        h = blk // BT_PER_W
        bti = blk - h * BT_PER_W
        pltpu.async_copy(table_hbm.at[idx_v.at[h, bti]], buf, sem)

    def drain_gather(buf, sem):
        pltpu.make_async_copy(table_hbm.at[pl.ds(0, 128)], buf, sem).wait()

    def drain_store(tbuf, sem):
        pltpu.make_async_copy(tbuf, out_hbm.at[0, :, 0], sem).wait()

    def handle(blk, buf, tbuf, sem_s, p):
        h = blk // BT_PER_W
        bti = blk - h * BT_PER_W

        @pl.when(p > 0)
        def _():
            drain_store(tbuf, sem_s)

        _transpose_block(buf, tbuf, iota16)
        pltpu.async_copy(tbuf, out_hbm.at[h, :, bt0 + bti], sem_s)

    fire(0, buf_a, sem_a)

    def body(p, _):
        blk = 2 * p
        fire(blk + 1, buf_b, sem_b)
        drain_gather(buf_a, sem_a)
        handle(blk, buf_a, tbuf_a, sem_sa, p)

        @pl.when(blk + 2 < NBLK)
        def _():
            fire(blk + 2, buf_a, sem_a)

        drain_gather(buf_b, sem_b)
        handle(blk + 1, buf_b, tbuf_b, sem_sb, p)
        return 0

    lax.fori_loop(0, NBLK // 2, body, 0)
    drain_store(tbuf_a, sem_sa)
    drain_store(tbuf_b, sem_sb)


@jax.jit
def _emb(xt, weight):
    mesh = plsc.VectorSubcoreMesh(core_axis_name="c", subcore_axis_name="s")
    f = pl.kernel(
        _emb_body,
        mesh=mesh,
        out_type=jax.ShapeDtypeStruct((HIST, D // 8, HB, 8, 128), jnp.float32),
        scratch_types=[
            pltpu.VMEM((HIST, BT_PER_W, 128), jnp.int32),
            pltpu.VMEM((128, D), jnp.float32),
            pltpu.VMEM((128, D), jnp.float32),
            pltpu.VMEM((D // 8, 8, 128), jnp.float32),
            pltpu.VMEM((D // 8, 8, 128), jnp.float32),
            pltpu.SemaphoreType.DMA,
            pltpu.SemaphoreType.DMA,
            pltpu.SemaphoreType.DMA,
            pltpu.SemaphoreType.DMA,
        ],
        compiler_params=pltpu.CompilerParams(
            use_tc_tiling_on_sc=False, needs_layout_passes=False),
    )
    return f(xt, weight)


def kernel(x, weight):
    xt = x.T.reshape(HIST, HB, 128)
    out = _emb(xt, weight)
    return out.transpose(2, 4, 0, 1, 3).reshape(BATCH, HIST, D)
